# baseline (speedup 1.0000x reference)
import functools

import jax
import jax.numpy as jnp
from jax import lax
from jax.experimental import pallas as pl
from jax.experimental.pallas import tpu as pltpu
from jax.experimental.pallas import tpu_sc as plsc

NC = 2
NS = 16
NW = NC * NS
EB = 128
KB = 8
BN = 1024
D = 128


def _pad_edges(src, dst, n_dst):
    e = src.shape[0]
    quantum = NW * EB * KB
    ep = -(-e // quantum) * quantum
    if ep > e:
        src = jnp.concatenate([src, jnp.zeros((ep - e,), jnp.int32)])
        dst = jnp.concatenate([dst, jnp.full((ep - e,), n_dst, jnp.int32)])
    return src.reshape(-1, EB), dst.reshape(-1, EB)


def _count(tilecnt, v):
    hi = lax.shift_right_logical(v, 7)
    lo = lax.bitwise_and(v, 127)
    rc, lastm = plsc.scan_count(v)
    plsc.addupdate_scatter(tilecnt, [hi, lo], rc.astype(jnp.float32), mask=lastm)


def _seg_sum(srcb, dstb, table, rng, nrange):
    nb = srcb.shape[0]
    tpb = nb // (NW * KB)
    n_pad = rng * nrange
    ZB = 32
    nzb_all = (rng + ZB) // ZB
    nrb_all = rng // EB
    zb_pt = -(-nzb_all // NS)
    rb_pt = -(-nrb_all // NS)
    f32, i32 = jnp.float32, jnp.int32

    mesh = plsc.VectorSubcoreMesh(core_axis_name="c", subcore_axis_name="s")

    @functools.partial(
        pl.kernel,
        out_type=jax.ShapeDtypeStruct((NC, n_pad, D), f32),
        mesh=mesh,
        compiler_params=pltpu.CompilerParams(needs_layout_passes=False),
        scratch_types=[
            pltpu.VMEM_SHARED((rng + ZB, D), f32),
            pltpu.VMEM((KB, EB), i32),
            pltpu.VMEM((KB, EB), i32),
            pltpu.VMEM((EB, D), f32),
            pltpu.VMEM((ZB, D), f32),
            pltpu.VMEM((1, EB), i32),
            pltpu.SemaphoreType.DMA,
        ],
    )
    def k(src_h, dst_h, tab_h, z_h, out_h,
          acc, sidx, didx, rows, zbuf, cbuf, sem):
        cid = lax.axis_index("c")
        sid = lax.axis_index("s")
        wid = sid * NC + cid
        pltpu.sync_copy(z_h.at[pl.ds(0, ZB)], zbuf)

        for r in range(nrange):
            base = r * rng

            def zbody(i, carry):
                b = sid * zb_pt + i

                @pl.when(b < nzb_all)
                def _():
                    pltpu.sync_copy(zbuf, acc.at[pl.ds(b * ZB, ZB)])
                return carry

            lax.fori_loop(0, zb_pt, zbody, 0)
            plsc.subcore_barrier()

            def ebody(i, carry):
                bb = wid * (tpb * KB) + i * KB
                pltpu.sync_copy(src_h.at[pl.ds(bb, KB)], sidx)
                pltpu.sync_copy(dst_h.at[pl.ds(bb, KB)], didx)
                for j in range(KB):
                    pltpu.async_copy(tab_h.at[sidx.at[j]], rows, sem).wait()
                    if nrange > 1:
                        for kk in range(EB // 16):
                            v = didx[j, pl.ds(kk * 16, 16)]
                            ld = v - base
                            m = (ld >= 0) & (ld < rng)
                            cbuf[0, pl.ds(kk * 16, 16)] = jnp.where(m, ld, rng)
                        pltpu.sync_copy(rows, acc.at[cbuf.at[0]], add=True)
                    else:
                        pltpu.sync_copy(rows, acc.at[didx.at[j]], add=True)
                return carry

            lax.fori_loop(0, tpb, ebody, 0)
            plsc.subcore_barrier()

            def rbody(i, carry):
                b = sid * rb_pt + i

                @pl.when(b < nrb_all)
                def _():
                    pltpu.sync_copy(acc.at[pl.ds(b * EB, EB)],
                                    out_h.at[cid].at[pl.ds(base + b * EB, EB)])
                return carry

            lax.fori_loop(0, rb_pt, rbody, 0)
            plsc.subcore_barrier()

    z = jnp.zeros((EB, D), f32)
    return k(srcb, dstb, table, z)


def _counts(dbs, chs):
    ZB = 32
    chmax = max(chs)
    f32, i32 = jnp.float32, jnp.int32
    mesh = plsc.VectorSubcoreMesh(core_axis_name="c", subcore_axis_name="s")

    @functools.partial(
        pl.kernel,
        out_type=[jax.ShapeDtypeStruct((NC, ch, EB), f32) for ch in chs],
        mesh=mesh,
        compiler_params=pltpu.CompilerParams(needs_layout_passes=False),
        scratch_types=(
            [pltpu.VMEM_SHARED((ch, EB), f32) for ch in chs]
            + [
                pltpu.VMEM((KB, EB), i32),
                pltpu.VMEM((chmax, EB), f32),
                pltpu.VMEM((ZB, EB), f32),
                pltpu.VMEM((1, EB), i32),
            ]
        ),
    )
    def k(*refs):
        ne = len(chs)
        d_hs, z_h = refs[:ne], refs[ne]
        cnt_hs = refs[ne + 1:2 * ne + 1]
        caccs = refs[2 * ne + 1:3 * ne + 1]
        didx, tilecnt, zbuf, fbuf = refs[3 * ne + 1:]
        cid = lax.axis_index("c")
        sid = lax.axis_index("s")
        wid = sid * NC + cid
        pltpu.sync_copy(z_h.at[pl.ds(0, ZB)], zbuf)

        for e in range(ne):
            ch = chs[e]
            d_h, cacc, cnt_h = d_hs[e], caccs[e], cnt_hs[e]
            nb = d_h.shape[0]
            tpb = nb // (NW * KB)
            for b in range(ch // EB):
                pltpu.sync_copy(z_h, tilecnt.at[pl.ds(b * EB, EB)])

            @pl.when(sid < ch // ZB)
            def _():
                pltpu.sync_copy(zbuf, cacc.at[pl.ds(sid * ZB, ZB)])
            plsc.subcore_barrier()

            def ebody(i, carry, d_h=d_h, tpb=tpb):
                bb = wid * (tpb * KB) + i * KB
                pltpu.sync_copy(d_h.at[pl.ds(bb, KB)], didx)
                for j in range(KB):
                    for kk in range(EB // 16):
                        _count(tilecnt, didx[j, pl.ds(kk * 16, 16)])
                return carry

            lax.fori_loop(0, tpb, ebody, 0)
            for b in range(ch // EB):
                for kk in range(EB // 16):
                    fbuf[0, pl.ds(kk * 16, 16)] = (
                        lax.iota(i32, 16) + (b * EB + kk * 16))
                pltpu.sync_copy(tilecnt.at[pl.ds(b * EB, EB)],
                                cacc.at[fbuf.at[0]], add=True)
            plsc.subcore_barrier()
            cpt = ch // NS
            sl = pl.ds(sid * cpt, cpt)
            pltpu.sync_copy(cacc.at[sl], cnt_h.at[cid].at[sl])

    z = jnp.zeros((EB, EB), f32)
    return k(*dbs, z)


def _mm(x, w, b):
    n = x.shape[0]
    f32 = jnp.float32

    def body(x_ref, w_ref, b_ref, o_ref):
        o_ref[...] = (jnp.dot(x_ref[...], w_ref[...],
                              preferred_element_type=f32) + b_ref[...])

    return pl.pallas_call(
        body,
        grid=(-(-n // BN),),
        in_specs=[
            pl.BlockSpec((BN, D), lambda i: (i, 0)),
            pl.BlockSpec((D, D), lambda i: (0, 0)),
            pl.BlockSpec((1, D), lambda i: (0, 0)),
        ],
        out_specs=pl.BlockSpec((BN, D), lambda i: (i, 0)),
        out_shape=jax.ShapeDtypeStruct((n, D), f32),
    )(x, w, b.reshape(1, D))


def _combine1(s1, cnt1, s3, cnt3, w2, b2, n):
    f32 = jnp.float32

    def body(s1r, s3r, c1r, c3r, w_ref, b_ref, o_ref):
        rsl = pl.ds(pl.program_id(0) * BN, BN)
        r1 = (1.0 / jnp.maximum(c1r[0, rsl] + c1r[1, rsl], 1.0))[:, None]
        r3 = (1.0 / jnp.maximum(c3r[0, rsl] + c3r[1, rsl], 1.0))[:, None]
        h = (s1r[0] + s1r[1]) * r1 + (s3r[0] + s3r[1]) * r3
        h = jnp.where(h >= 0.0, h, 0.01 * h)
        o_ref[...] = (jnp.dot(h, w_ref[...],
                              preferred_element_type=f32) + b_ref[...])

    spec_s = pl.BlockSpec((NC, BN, D), lambda i: (0, i, 0))

    def spec_c(cnt):
        return pl.BlockSpec((NC, cnt.shape[1]), lambda i: (0, 0))

    return pl.pallas_call(
        body,
        grid=(n // BN,),
        in_specs=[spec_s, spec_s, spec_c(cnt1), spec_c(cnt3),
                  pl.BlockSpec((D, D), lambda i: (0, 0)),
                  pl.BlockSpec((1, D), lambda i: (0, 0))],
        out_specs=pl.BlockSpec((BN, D), lambda i: (i, 0)),
        out_shape=jax.ShapeDtypeStruct((n, D), f32),
    )(s1, s3, cnt1, cnt3, w2, b2.reshape(1, D))


def _combine2(s2, cnt2, w_fc, b_fc, n):
    f32 = jnp.float32
    dout = w_fc.shape[1]

    def body(s_ref, c_ref, w_ref, b_ref, o_ref, p_ref):
        rsl = pl.ds(pl.program_id(0) * BN, BN)
        r = (1.0 / jnp.maximum(c_ref[0, rsl] + c_ref[1, rsl], 1.0))[:, None]
        o = (s_ref[0] + s_ref[1]) * r
        o_ref[...] = o
        p_ref[...] = (jnp.dot(o, w_ref[...],
                              preferred_element_type=f32) + b_ref[...])

    return pl.pallas_call(
        body,
        grid=(-(-n // BN),),
        in_specs=[
            pl.BlockSpec((NC, BN, D), lambda i: (0, i, 0)),
            pl.BlockSpec((NC, cnt2.shape[1]), lambda i: (0, 0)),
            pl.BlockSpec((D, dout), lambda i: (0, 0)),
            pl.BlockSpec((1, dout), lambda i: (0, 0)),
        ],
        out_specs=[
            pl.BlockSpec((BN, D), lambda i: (i, 0)),
            pl.BlockSpec((BN, dout), lambda i: (i, 0)),
        ],
        out_shape=[
            jax.ShapeDtypeStruct((n, D), f32),
            jax.ShapeDtypeStruct((n, dout), f32),
        ],
    )(s2, cnt2, w_fc, b_fc.reshape(1, dout))


def kernel(x_ent, x_cls, e1_src, e1_dst, e2_src, e2_dst, e3_src, e3_dst,
           W1_e1, b1_e1, W1_e2, b1_e2, W1_e3, b1_e3,
           W2_e1, b2_e1, W2_e2, b2_e2, W2_e3, b2_e3,
           W_fc, b_fc):
    n_ent = x_ent.shape[0]
    n_cls = x_cls.shape[0]
    rng_e, nr_e, ch_e = 12544, 4, 512
    rng_c, nr_c, ch_c = 10240, 1, 128

    t1 = _mm(x_ent, W1_e1, b1_e1)
    t3 = _mm(x_cls, W1_e3, b1_e3)

    s1b, d1b = _pad_edges(e1_src, e1_dst, n_ent)
    s3b, d3b = _pad_edges(e3_src, e3_dst, n_ent)
    s2b, d2b = _pad_edges(e2_src, e2_dst, n_cls)
    cnt1, cnt3, cnt2 = _counts([d1b, d3b, d2b], [ch_e, ch_e, ch_c])

    s1 = _seg_sum(s1b, d1b, t1, rng_e, nr_e)
    s3 = _seg_sum(s3b, d3b, t3, rng_e, nr_e)

    t2 = _combine1(s1, cnt1.reshape(NC, ch_e * EB),
                   s3, cnt3.reshape(NC, ch_e * EB), W2_e2, b2_e2,
                   rng_e * nr_e)

    s2 = _seg_sum(s2b, d2b, t2, rng_c, nr_c)

    output, pred = _combine2(s2, cnt2.reshape(NC, ch_c * EB), W_fc, b_fc, n_cls)
    return (output, pred)


# speedup vs baseline: 1.1543x; 1.1543x over previous
import functools

import jax
import jax.numpy as jnp
from jax import lax
from jax.experimental import pallas as pl
from jax.experimental.pallas import tpu as pltpu
from jax.experimental.pallas import tpu_sc as plsc

NC = 2
NS = 16
NW = NC * NS
EB = 128
KB = 8
BN = 1024
D = 128


def _pad_edges(src, dst, n_dst):
    e = src.shape[0]
    quantum = NW * EB * KB
    ep = -(-e // quantum) * quantum
    if ep > e:
        src = jnp.concatenate([src, jnp.zeros((ep - e,), jnp.int32)])
        dst = jnp.concatenate([dst, jnp.full((ep - e,), n_dst, jnp.int32)])
    return src.reshape(-1, EB), dst.reshape(-1, EB)


def _count(tilecnt, v):
    hi = lax.shift_right_logical(v, 7)
    lo = lax.bitwise_and(v, 127)
    rc, lastm = plsc.scan_count(v)
    plsc.addupdate_scatter(tilecnt, [hi, lo], rc.astype(jnp.float32), mask=lastm)


def _seg_sum(srcb, dstb, table, rng, nrange):
    GB = 64
    srcb = srcb.reshape(-1, GB)
    dstb = dstb.reshape(-1, GB)
    nb = srcb.shape[0]
    tpb = nb // (NW * KB)
    n_pad = rng * nrange
    ZB = 16
    nzb_all = (rng + ZB) // ZB
    nrb_all = rng // EB
    zb_pt = -(-nzb_all // NS)
    rb_pt = -(-nrb_all // NS)
    f32, i32 = jnp.float32, jnp.int32

    mesh = plsc.VectorSubcoreMesh(core_axis_name="c", subcore_axis_name="s")

    @functools.partial(
        pl.kernel,
        out_type=jax.ShapeDtypeStruct((NC, n_pad, D), f32),
        mesh=mesh,
        compiler_params=pltpu.CompilerParams(needs_layout_passes=False),
        scratch_types=[
            pltpu.VMEM_SHARED((rng + ZB, D), f32),
            pltpu.VMEM((KB, GB), i32),
            pltpu.VMEM((KB, GB), i32),
            pltpu.VMEM((GB, D), f32),
            pltpu.VMEM((GB, D), f32),
            pltpu.VMEM((GB, D), f32),
            pltpu.VMEM((ZB, D), f32),
            pltpu.VMEM((1, GB), i32),
            pltpu.VMEM((1, GB), i32),
            pltpu.VMEM((1, GB), i32),
            pltpu.SemaphoreType.DMA,
            pltpu.SemaphoreType.DMA,
        ],
    )
    def k(src_h, dst_h, tab_h, z_h, out_h,
          acc, sidx, didx, r0, r1, r2, zbuf, c0, c1, c2, sem_g, sem_s):
        rowb = (r0, r1, r2)
        cbufs = (c0, c1, c2)
        cid = lax.axis_index("c")
        sid = lax.axis_index("s")
        wid = sid * NC + cid
        pltpu.sync_copy(z_h.at[pl.ds(0, ZB)], zbuf)

        def scatter_idx(j, base):
            if nrange > 1:
                cb = cbufs[j % 3]
                for kk in range(GB // 16):
                    v = didx[j, pl.ds(kk * 16, 16)]
                    ld = v - base
                    m = (ld >= 0) & (ld < rng)
                    cb[0, pl.ds(kk * 16, 16)] = jnp.where(m, ld, rng)
                return cb.at[0]
            return didx.at[j]

        for r in range(nrange):
            base = r * rng

            def zbody(i, carry):
                b = sid * zb_pt + i

                @pl.when(b < nzb_all)
                def _():
                    pltpu.sync_copy(zbuf, acc.at[pl.ds(b * ZB, ZB)])
                return carry

            lax.fori_loop(0, zb_pt, zbody, 0)
            plsc.subcore_barrier()

            def ebody(i, carry, base=base):
                bb = wid * (tpb * KB) + i * KB
                pltpu.sync_copy(src_h.at[pl.ds(bb, KB)], sidx)
                pltpu.sync_copy(dst_h.at[pl.ds(bb, KB)], didx)
                pltpu.async_copy(tab_h.at[sidx.at[0]], rowb[0], sem_g)
                pltpu.async_copy(tab_h.at[sidx.at[1]], rowb[1], sem_g)
                idxs = [None] * KB
                for j in range(KB):
                    rb = rowb[j % 3]
                    pltpu.make_async_copy(tab_h.at[sidx.at[j]], rb,
                                          sem_g).wait()
                    idxs[j] = scatter_idx(j, base)
                    pltpu.async_copy(rb, acc.at[idxs[j]], sem_s, add=True)
                    if j >= 1:
                        pb = rowb[(j - 1) % 3]
                        pltpu.make_async_copy(pb, acc.at[idxs[j - 1]],
                                              sem_s).wait()
                    if j + 2 < KB:
                        pltpu.async_copy(tab_h.at[sidx.at[j + 2]],
                                         rowb[(j + 2) % 3], sem_g)
                pltpu.make_async_copy(rowb[(KB - 1) % 3], acc.at[idxs[KB - 1]],
                                      sem_s).wait()
                return carry

            lax.fori_loop(0, tpb, ebody, 0)
            plsc.subcore_barrier()

            def rbody(i, carry, base=base):
                b = sid * rb_pt + i

                @pl.when(b < nrb_all)
                def _():
                    pltpu.sync_copy(acc.at[pl.ds(b * EB, EB)],
                                    out_h.at[cid].at[pl.ds(base + b * EB, EB)])
                return carry

            lax.fori_loop(0, rb_pt, rbody, 0)
            plsc.subcore_barrier()

    z = jnp.zeros((EB, D), f32)
    return k(srcb, dstb, table, z)


def _counts(dbs, chs):
    ZB = 32
    chmax = max(chs)
    f32, i32 = jnp.float32, jnp.int32
    mesh = plsc.VectorSubcoreMesh(core_axis_name="c", subcore_axis_name="s")

    @functools.partial(
        pl.kernel,
        out_type=[jax.ShapeDtypeStruct((NC, ch, EB), f32) for ch in chs],
        mesh=mesh,
        compiler_params=pltpu.CompilerParams(needs_layout_passes=False),
        scratch_types=(
            [pltpu.VMEM_SHARED((ch, EB), f32) for ch in chs]
            + [
                pltpu.VMEM((KB, EB), i32),
                pltpu.VMEM((chmax, EB), f32),
                pltpu.VMEM((ZB, EB), f32),
                pltpu.VMEM((1, EB), i32),
            ]
        ),
    )
    def k(*refs):
        ne = len(chs)
        d_hs, z_h = refs[:ne], refs[ne]
        cnt_hs = refs[ne + 1:2 * ne + 1]
        caccs = refs[2 * ne + 1:3 * ne + 1]
        didx, tilecnt, zbuf, fbuf = refs[3 * ne + 1:]
        cid = lax.axis_index("c")
        sid = lax.axis_index("s")
        wid = sid * NC + cid
        pltpu.sync_copy(z_h.at[pl.ds(0, ZB)], zbuf)

        for e in range(ne):
            ch = chs[e]
            d_h, cacc, cnt_h = d_hs[e], caccs[e], cnt_hs[e]
            nb = d_h.shape[0]
            tpb = nb // (NW * KB)
            for b in range(ch // EB):
                pltpu.sync_copy(z_h, tilecnt.at[pl.ds(b * EB, EB)])

            @pl.when(sid < ch // ZB)
            def _():
                pltpu.sync_copy(zbuf, cacc.at[pl.ds(sid * ZB, ZB)])
            plsc.subcore_barrier()

            def ebody(i, carry, d_h=d_h, tpb=tpb):
                bb = wid * (tpb * KB) + i * KB
                pltpu.sync_copy(d_h.at[pl.ds(bb, KB)], didx)
                for j in range(KB):
                    for kk in range(EB // 16):
                        _count(tilecnt, didx[j, pl.ds(kk * 16, 16)])
                return carry

            lax.fori_loop(0, tpb, ebody, 0)
            for b in range(ch // EB):
                for kk in range(EB // 16):
                    fbuf[0, pl.ds(kk * 16, 16)] = (
                        lax.iota(i32, 16) + (b * EB + kk * 16))
                pltpu.sync_copy(tilecnt.at[pl.ds(b * EB, EB)],
                                cacc.at[fbuf.at[0]], add=True)
            plsc.subcore_barrier()
            cpt = ch // NS
            sl = pl.ds(sid * cpt, cpt)
            pltpu.sync_copy(cacc.at[sl], cnt_h.at[cid].at[sl])

    z = jnp.zeros((EB, EB), f32)
    return k(*dbs, z)


def _mm(x, w, b):
    n = x.shape[0]
    f32 = jnp.float32

    def body(x_ref, w_ref, b_ref, o_ref):
        o_ref[...] = (jnp.dot(x_ref[...], w_ref[...],
                              preferred_element_type=f32) + b_ref[...])

    return pl.pallas_call(
        body,
        grid=(-(-n // BN),),
        in_specs=[
            pl.BlockSpec((BN, D), lambda i: (i, 0)),
            pl.BlockSpec((D, D), lambda i: (0, 0)),
            pl.BlockSpec((1, D), lambda i: (0, 0)),
        ],
        out_specs=pl.BlockSpec((BN, D), lambda i: (i, 0)),
        out_shape=jax.ShapeDtypeStruct((n, D), f32),
    )(x, w, b.reshape(1, D))


def _combine1(s1, cnt1, s3, cnt3, w2, b2, n):
    f32 = jnp.float32

    def body(s1r, s3r, c1r, c3r, w_ref, b_ref, o_ref):
        rsl = pl.ds(pl.program_id(0) * BN, BN)
        r1 = (1.0 / jnp.maximum(c1r[0, rsl] + c1r[1, rsl], 1.0))[:, None]
        r3 = (1.0 / jnp.maximum(c3r[0, rsl] + c3r[1, rsl], 1.0))[:, None]
        h = (s1r[0] + s1r[1]) * r1 + (s3r[0] + s3r[1]) * r3
        h = jnp.where(h >= 0.0, h, 0.01 * h)
        o_ref[...] = (jnp.dot(h, w_ref[...],
                              preferred_element_type=f32) + b_ref[...])

    spec_s = pl.BlockSpec((NC, BN, D), lambda i: (0, i, 0))

    def spec_c(cnt):
        return pl.BlockSpec((NC, cnt.shape[1]), lambda i: (0, 0))

    return pl.pallas_call(
        body,
        grid=(n // BN,),
        in_specs=[spec_s, spec_s, spec_c(cnt1), spec_c(cnt3),
                  pl.BlockSpec((D, D), lambda i: (0, 0)),
                  pl.BlockSpec((1, D), lambda i: (0, 0))],
        out_specs=pl.BlockSpec((BN, D), lambda i: (i, 0)),
        out_shape=jax.ShapeDtypeStruct((n, D), f32),
    )(s1, s3, cnt1, cnt3, w2, b2.reshape(1, D))


def _combine2(s2, cnt2, w_fc, b_fc, n):
    f32 = jnp.float32
    dout = w_fc.shape[1]

    def body(s_ref, c_ref, w_ref, b_ref, o_ref, p_ref):
        rsl = pl.ds(pl.program_id(0) * BN, BN)
        r = (1.0 / jnp.maximum(c_ref[0, rsl] + c_ref[1, rsl], 1.0))[:, None]
        o = (s_ref[0] + s_ref[1]) * r
        o_ref[...] = o
        p_ref[...] = (jnp.dot(o, w_ref[...],
                              preferred_element_type=f32) + b_ref[...])

    return pl.pallas_call(
        body,
        grid=(-(-n // BN),),
        in_specs=[
            pl.BlockSpec((NC, BN, D), lambda i: (0, i, 0)),
            pl.BlockSpec((NC, cnt2.shape[1]), lambda i: (0, 0)),
            pl.BlockSpec((D, dout), lambda i: (0, 0)),
            pl.BlockSpec((1, dout), lambda i: (0, 0)),
        ],
        out_specs=[
            pl.BlockSpec((BN, D), lambda i: (i, 0)),
            pl.BlockSpec((BN, dout), lambda i: (i, 0)),
        ],
        out_shape=[
            jax.ShapeDtypeStruct((n, D), f32),
            jax.ShapeDtypeStruct((n, dout), f32),
        ],
    )(s2, cnt2, w_fc, b_fc.reshape(1, dout))


def kernel(x_ent, x_cls, e1_src, e1_dst, e2_src, e2_dst, e3_src, e3_dst,
           W1_e1, b1_e1, W1_e2, b1_e2, W1_e3, b1_e3,
           W2_e1, b2_e1, W2_e2, b2_e2, W2_e3, b2_e3,
           W_fc, b_fc):
    n_ent = x_ent.shape[0]
    n_cls = x_cls.shape[0]
    rng_e, nr_e, ch_e = 12544, 4, 512
    rng_c, nr_c, ch_c = 10240, 1, 128

    t1 = _mm(x_ent, W1_e1, b1_e1)
    t3 = _mm(x_cls, W1_e3, b1_e3)

    s1b, d1b = _pad_edges(e1_src, e1_dst, n_ent)
    s3b, d3b = _pad_edges(e3_src, e3_dst, n_ent)
    s2b, d2b = _pad_edges(e2_src, e2_dst, n_cls)
    cnt1, cnt3, cnt2 = _counts([d1b, d3b, d2b], [ch_e, ch_e, ch_c])

    s1 = _seg_sum(s1b, d1b, t1, rng_e, nr_e)
    s3 = _seg_sum(s3b, d3b, t3, rng_e, nr_e)

    t2 = _combine1(s1, cnt1.reshape(NC, ch_e * EB),
                   s3, cnt3.reshape(NC, ch_e * EB), W2_e2, b2_e2,
                   rng_e * nr_e)

    s2 = _seg_sum(s2b, d2b, t2, rng_c, nr_c)

    output, pred = _combine2(s2, cnt2.reshape(NC, ch_c * EB), W_fc, b_fc, n_cls)
    return (output, pred)


# speedup vs baseline: 2.1902x; 1.8973x over previous
import functools

import jax
import jax.numpy as jnp
from jax import lax
from jax.experimental import pallas as pl
from jax.experimental.pallas import tpu as pltpu
from jax.experimental.pallas import tpu_sc as plsc

NC = 2
NS = 16
NW = NC * NS
EB = 128
KB = 8
BN = 1024
D = 128


def _pad_edges(src, dst, n_dst):
    e = src.shape[0]
    quantum = NW * EB * KB
    ep = -(-e // quantum) * quantum
    if ep > e:
        src = jnp.concatenate([src, jnp.zeros((ep - e,), jnp.int32)])
        dst = jnp.concatenate([dst, jnp.full((ep - e,), n_dst, jnp.int32)])
    return src.reshape(-1, EB), dst.reshape(-1, EB)


def _count(tilecnt, v):
    hi = lax.shift_right_logical(v, 7)
    lo = lax.bitwise_and(v, 127)
    rc, lastm = plsc.scan_count(v)
    plsc.addupdate_scatter(tilecnt, [hi, lo], rc.astype(jnp.float32), mask=lastm)


def _seg_sum(srcb, dstb, table, rng, nrange):
    GB = 64
    srcb = srcb.reshape(-1, GB)
    dstb = dstb.reshape(-1, GB)
    nb = srcb.shape[0]
    tpb = nb // (NW * KB)
    n_pad = rng * nrange
    ZB = 16
    nzb_all = (rng + ZB) // ZB
    nrb_all = rng // EB
    zb_pt = -(-nzb_all // NS)
    rb_pt = -(-nrb_all // NS)
    f32, i32 = jnp.float32, jnp.int32

    mesh = plsc.VectorSubcoreMesh(core_axis_name="c", subcore_axis_name="s")

    @functools.partial(
        pl.kernel,
        out_type=jax.ShapeDtypeStruct((NC, n_pad, D), f32),
        mesh=mesh,
        compiler_params=pltpu.CompilerParams(needs_layout_passes=False),
        scratch_types=[
            pltpu.VMEM_SHARED((rng + ZB, D), f32),
            pltpu.VMEM((KB, GB), i32),
            pltpu.VMEM((KB, GB), i32),
            pltpu.VMEM((GB, D), f32),
            pltpu.VMEM((GB, D), f32),
            pltpu.VMEM((GB, D), f32),
            pltpu.VMEM((ZB, D), f32),
            pltpu.VMEM((1, GB), i32),
            pltpu.VMEM((1, GB), i32),
            pltpu.VMEM((1, GB), i32),
            pltpu.SemaphoreType.DMA,
            pltpu.SemaphoreType.DMA,
        ],
    )
    def k(src_h, dst_h, tab_h, z_h, out_h,
          acc, sidx, didx, r0, r1, r2, zbuf, c0, c1, c2, sem_g, sem_s):
        rowb = (r0, r1, r2)
        cbufs = (c0, c1, c2)
        cid = lax.axis_index("c")
        sid = lax.axis_index("s")
        wid = sid * NC + cid
        pltpu.sync_copy(z_h.at[pl.ds(0, ZB)], zbuf)

        def scatter_idx(j, base):
            if nrange > 1:
                cb = cbufs[j % 3]
                for kk in range(GB // 16):
                    v = didx[j, pl.ds(kk * 16, 16)]
                    ld = v - base
                    m = (ld >= 0) & (ld < rng)
                    cb[0, pl.ds(kk * 16, 16)] = jnp.where(m, ld, rng)
                return cb.at[0]
            return didx.at[j]

        for r in range(nrange):
            base = r * rng

            def zbody(i, carry):
                b = sid * zb_pt + i

                @pl.when(b < nzb_all)
                def _():
                    pltpu.sync_copy(zbuf, acc.at[pl.ds(b * ZB, ZB)])
                return carry

            lax.fori_loop(0, zb_pt, zbody, 0)
            plsc.subcore_barrier()

            def ebody(i, carry, base=base):
                bb = wid * (tpb * KB) + i * KB
                pltpu.sync_copy(src_h.at[pl.ds(bb, KB)], sidx)
                pltpu.sync_copy(dst_h.at[pl.ds(bb, KB)], didx)
                pltpu.async_copy(tab_h.at[sidx.at[0]], rowb[0], sem_g)
                pltpu.async_copy(tab_h.at[sidx.at[1]], rowb[1], sem_g)
                idxs = [None] * KB
                for j in range(KB):
                    rb = rowb[j % 3]
                    pltpu.make_async_copy(tab_h.at[sidx.at[j]], rb,
                                          sem_g).wait()
                    idxs[j] = scatter_idx(j, base)
                    pltpu.async_copy(rb, acc.at[idxs[j]], sem_s, add=True)
                    if j >= 1:
                        pb = rowb[(j - 1) % 3]
                        pltpu.make_async_copy(pb, acc.at[idxs[j - 1]],
                                              sem_s).wait()
                    if j + 2 < KB:
                        pltpu.async_copy(tab_h.at[sidx.at[j + 2]],
                                         rowb[(j + 2) % 3], sem_g)
                pltpu.make_async_copy(rowb[(KB - 1) % 3], acc.at[idxs[KB - 1]],
                                      sem_s).wait()
                return carry

            lax.fori_loop(0, tpb, ebody, 0)
            plsc.subcore_barrier()

            def rbody(i, carry, base=base):
                b = sid * rb_pt + i

                @pl.when(b < nrb_all)
                def _():
                    pltpu.sync_copy(acc.at[pl.ds(b * EB, EB)],
                                    out_h.at[cid].at[pl.ds(base + b * EB, EB)])
                return carry

            lax.fori_loop(0, rb_pt, rbody, 0)
            plsc.subcore_barrier()

    z = jnp.zeros((EB, D), f32)
    return k(srcb, dstb, table, z)


def _bucket(srcb, dstb, rng, nrange):
    GB = 128
    srcb = srcb.reshape(-1, GB)
    dstb = dstb.reshape(-1, GB)
    nb = srcb.shape[0]
    tpb = nb // (NW * KB)
    nbt = tpb * KB
    capb = nbt + 1
    i32 = jnp.int32
    mesh = plsc.VectorSubcoreMesh(core_axis_name="c", subcore_axis_name="s")

    @functools.partial(
        pl.kernel,
        out_type=[jax.ShapeDtypeStruct((NW, nrange, capb, 2, GB), i32),
                  jax.ShapeDtypeStruct((NW, 16), i32)],
        mesh=mesh,
        compiler_params=pltpu.CompilerParams(needs_layout_passes=False),
        scratch_types=(
            [pltpu.VMEM((KB, GB), i32), pltpu.VMEM((KB, GB), i32)]
            + [pltpu.VMEM((2 * GB,), i32) for _ in range(2 * nrange)]
            + [pltpu.VMEM((1, 16), i32)]
        ),
    )
    def k(src_h, dst_h, ob_h, sz_h, sidx, didx, *rest):
        pend = rest[:2 * nrange]
        szbuf = rest[2 * nrange]
        cid = lax.axis_index("c")
        sid = lax.axis_index("s")
        wid = sid * NC + cid
        zero = jnp.zeros((), i32)

        def ebody(i, carry):
            bb = wid * nbt + i * KB
            pltpu.sync_copy(src_h.at[pl.ds(bb, KB)], sidx)
            pltpu.sync_copy(dst_h.at[pl.ds(bb, KB)], didx)
            st = list(carry)
            for j in range(KB):
                for r in range(nrange):
                    ps, pd = pend[2 * r], pend[2 * r + 1]
                    cur = st[r]
                    for kk in range(GB // 16):
                        sv = sidx[j, pl.ds(kk * 16, 16)]
                        v = didx[j, pl.ds(kk * 16, 16)]
                        m = (v >= r * rng) & (v < (r + 1) * rng)
                        mi = m.astype(i32)
                        pos = cur + plsc.cumsum(mi) - 1
                        plsc.store_scatter(ps, [pos], sv, mask=m)
                        plsc.store_scatter(pd, [pos], v - r * rng, mask=m)
                        cur = cur + jnp.sum(mi)
                    nf = st[nrange + r]

                    @pl.when(cur >= GB)
                    def _(ps=ps, pd=pd, nf=nf, r=r):
                        pltpu.sync_copy(
                            ps.at[pl.ds(0, GB)],
                            ob_h.at[wid].at[r].at[nf].at[0])
                        pltpu.sync_copy(
                            pd.at[pl.ds(0, GB)],
                            ob_h.at[wid].at[r].at[nf].at[1])
                        for kk in range(GB // 16):
                            ps[pl.ds(kk * 16, 16)] = ps[pl.ds(GB + kk * 16, 16)]
                            pd[pl.ds(kk * 16, 16)] = pd[pl.ds(GB + kk * 16, 16)]

                    flushed = cur >= GB
                    st[r] = jnp.where(flushed, cur - GB, cur)
                    st[nrange + r] = jnp.where(flushed, nf + 1, nf)
            return tuple(st)

        carry = tuple(zero for _ in range(2 * nrange))
        carry = lax.fori_loop(0, tpb, ebody, carry)

        szv = jnp.zeros((16,), i32)
        iot = lax.iota(i32, 16)
        for r in range(nrange):
            ps, pd = pend[2 * r], pend[2 * r + 1]
            cur, nf = carry[r], carry[nrange + r]
            for kk in range(GB // 16):
                p = iot + kk * 16
                m = (p >= cur) & (p < GB)
                plsc.store_scatter(ps, [p], jnp.zeros((16,), i32), mask=m)
                plsc.store_scatter(pd, [p], jnp.full((16,), rng, i32), mask=m)
            do_flush = (cur > 0) | (nf == 0)

            @pl.when(do_flush)
            def _(ps=ps, pd=pd, nf=nf, r=r):
                pltpu.sync_copy(ps.at[pl.ds(0, GB)],
                                ob_h.at[wid].at[r].at[nf].at[0])
                pltpu.sync_copy(pd.at[pl.ds(0, GB)],
                                ob_h.at[wid].at[r].at[nf].at[1])

            nf_final = jnp.where(do_flush, nf + 1, nf)
            szv = jnp.where(iot == r, nf_final, szv)
        szbuf[0, pl.ds(0, 16)] = szv
        pltpu.sync_copy(szbuf.at[0], sz_h.at[wid])

    return k(srcb, dstb)


def _seg_sum_b(obuf, sizes, table, rng, nrange):
    GB = 128
    capb = obuf.shape[2]
    n_pad = rng * nrange
    ZB = 16
    nzb_all = (rng + ZB) // ZB
    nrb_all = rng // EB
    zb_pt = -(-nzb_all // NS)
    rb_pt = -(-nrb_all // NS)
    f32, i32 = jnp.float32, jnp.int32
    mesh = plsc.VectorSubcoreMesh(core_axis_name="c", subcore_axis_name="s")

    @functools.partial(
        pl.kernel,
        out_type=jax.ShapeDtypeStruct((NC, n_pad, D), f32),
        mesh=mesh,
        compiler_params=pltpu.CompilerParams(needs_layout_passes=False),
        scratch_types=[
            pltpu.VMEM_SHARED((rng + ZB, D), f32),
            pltpu.VMEM((2, GB), i32),
            pltpu.VMEM((GB, D), f32),
            pltpu.VMEM((ZB, D), f32),
            pltpu.VMEM((1, 16), i32),
            pltpu.SemaphoreType.DMA,
        ],
    )
    def k(ob_h, sz_h, tab_h, z_h, out_h, acc, sd, rows, zbuf, szbuf, sem):
        cid = lax.axis_index("c")
        sid = lax.axis_index("s")
        wid = sid * NC + cid
        pltpu.sync_copy(z_h.at[pl.ds(0, ZB)], zbuf)
        pltpu.sync_copy(sz_h.at[wid], szbuf.at[0])

        for r in range(nrange):
            base = r * rng

            def zbody(i, carry):
                b = sid * zb_pt + i

                @pl.when(b < nzb_all)
                def _():
                    pltpu.sync_copy(zbuf, acc.at[pl.ds(b * ZB, ZB)])
                return carry

            lax.fori_loop(0, zb_pt, zbody, 0)
            plsc.subcore_barrier()

            szv = szbuf[0, pl.ds(0, 16)]
            nfr = jnp.sum(jnp.where(lax.iota(i32, 16) == r, szv, 0))

            def ebody(i, carry, r=r):
                @pl.when(i < nfr)
                def _():
                    pltpu.sync_copy(ob_h.at[wid].at[r].at[i], sd)
                    pltpu.async_copy(tab_h.at[sd.at[0]], rows, sem).wait()
                    pltpu.sync_copy(rows, acc.at[sd.at[1]], add=True)
                return carry

            lax.fori_loop(0, capb, ebody, 0)
            plsc.subcore_barrier()

            def rbody(i, carry, base=base):
                b = sid * rb_pt + i

                @pl.when(b < nrb_all)
                def _():
                    pltpu.sync_copy(acc.at[pl.ds(b * EB, EB)],
                                    out_h.at[cid].at[pl.ds(base + b * EB, EB)])
                return carry

            lax.fori_loop(0, rb_pt, rbody, 0)
            plsc.subcore_barrier()

    z = jnp.zeros((EB, D), f32)
    return k(obuf, sizes, table, z)


def _counts(dbs, chs):
    ZB = 32
    chmax = max(chs)
    f32, i32 = jnp.float32, jnp.int32
    mesh = plsc.VectorSubcoreMesh(core_axis_name="c", subcore_axis_name="s")

    @functools.partial(
        pl.kernel,
        out_type=[jax.ShapeDtypeStruct((NC, ch, EB), f32) for ch in chs],
        mesh=mesh,
        compiler_params=pltpu.CompilerParams(needs_layout_passes=False),
        scratch_types=(
            [pltpu.VMEM_SHARED((ch, EB), f32) for ch in chs]
            + [
                pltpu.VMEM((KB, EB), i32),
                pltpu.VMEM((chmax, EB), f32),
                pltpu.VMEM((ZB, EB), f32),
                pltpu.VMEM((1, EB), i32),
            ]
        ),
    )
    def k(*refs):
        ne = len(chs)
        d_hs, z_h = refs[:ne], refs[ne]
        cnt_hs = refs[ne + 1:2 * ne + 1]
        caccs = refs[2 * ne + 1:3 * ne + 1]
        didx, tilecnt, zbuf, fbuf = refs[3 * ne + 1:]
        cid = lax.axis_index("c")
        sid = lax.axis_index("s")
        wid = sid * NC + cid
        pltpu.sync_copy(z_h.at[pl.ds(0, ZB)], zbuf)

        for e in range(ne):
            ch = chs[e]
            d_h, cacc, cnt_h = d_hs[e], caccs[e], cnt_hs[e]
            nb = d_h.shape[0]
            tpb = nb // (NW * KB)
            for b in range(ch // EB):
                pltpu.sync_copy(z_h, tilecnt.at[pl.ds(b * EB, EB)])

            @pl.when(sid < ch // ZB)
            def _():
                pltpu.sync_copy(zbuf, cacc.at[pl.ds(sid * ZB, ZB)])
            plsc.subcore_barrier()

            def ebody(i, carry, d_h=d_h, tpb=tpb):
                bb = wid * (tpb * KB) + i * KB
                pltpu.sync_copy(d_h.at[pl.ds(bb, KB)], didx)
                for j in range(KB):
                    for kk in range(EB // 16):
                        _count(tilecnt, didx[j, pl.ds(kk * 16, 16)])
                return carry

            lax.fori_loop(0, tpb, ebody, 0)
            for b in range(ch // EB):
                for kk in range(EB // 16):
                    fbuf[0, pl.ds(kk * 16, 16)] = (
                        lax.iota(i32, 16) + (b * EB + kk * 16))
                pltpu.sync_copy(tilecnt.at[pl.ds(b * EB, EB)],
                                cacc.at[fbuf.at[0]], add=True)
            plsc.subcore_barrier()
            cpt = ch // NS
            sl = pl.ds(sid * cpt, cpt)
            pltpu.sync_copy(cacc.at[sl], cnt_h.at[cid].at[sl])

    z = jnp.zeros((EB, EB), f32)
    return k(*dbs, z)


def _mm(x, w, b):
    n = x.shape[0]
    f32 = jnp.float32

    def body(x_ref, w_ref, b_ref, o_ref):
        o_ref[...] = (jnp.dot(x_ref[...], w_ref[...],
                              preferred_element_type=f32) + b_ref[...])

    return pl.pallas_call(
        body,
        grid=(-(-n // BN),),
        in_specs=[
            pl.BlockSpec((BN, D), lambda i: (i, 0)),
            pl.BlockSpec((D, D), lambda i: (0, 0)),
            pl.BlockSpec((1, D), lambda i: (0, 0)),
        ],
        out_specs=pl.BlockSpec((BN, D), lambda i: (i, 0)),
        out_shape=jax.ShapeDtypeStruct((n, D), f32),
    )(x, w, b.reshape(1, D))


def _combine1(s1, cnt1, s3, cnt3, w2, b2, n):
    f32 = jnp.float32

    def body(s1r, s3r, c1r, c3r, w_ref, b_ref, o_ref):
        rsl = pl.ds(pl.program_id(0) * BN, BN)
        r1 = (1.0 / jnp.maximum(c1r[0, rsl] + c1r[1, rsl], 1.0))[:, None]
        r3 = (1.0 / jnp.maximum(c3r[0, rsl] + c3r[1, rsl], 1.0))[:, None]
        h = (s1r[0] + s1r[1]) * r1 + (s3r[0] + s3r[1]) * r3
        h = jnp.where(h >= 0.0, h, 0.01 * h)
        o_ref[...] = (jnp.dot(h, w_ref[...],
                              preferred_element_type=f32) + b_ref[...])

    spec_s = pl.BlockSpec((NC, BN, D), lambda i: (0, i, 0))

    def spec_c(cnt):
        return pl.BlockSpec((NC, cnt.shape[1]), lambda i: (0, 0))

    return pl.pallas_call(
        body,
        grid=(n // BN,),
        in_specs=[spec_s, spec_s, spec_c(cnt1), spec_c(cnt3),
                  pl.BlockSpec((D, D), lambda i: (0, 0)),
                  pl.BlockSpec((1, D), lambda i: (0, 0))],
        out_specs=pl.BlockSpec((BN, D), lambda i: (i, 0)),
        out_shape=jax.ShapeDtypeStruct((n, D), f32),
    )(s1, s3, cnt1, cnt3, w2, b2.reshape(1, D))


def _combine2(s2, cnt2, w_fc, b_fc, n):
    f32 = jnp.float32
    dout = w_fc.shape[1]

    def body(s_ref, c_ref, w_ref, b_ref, o_ref, p_ref):
        rsl = pl.ds(pl.program_id(0) * BN, BN)
        r = (1.0 / jnp.maximum(c_ref[0, rsl] + c_ref[1, rsl], 1.0))[:, None]
        o = (s_ref[0] + s_ref[1]) * r
        o_ref[...] = o
        p_ref[...] = (jnp.dot(o, w_ref[...],
                              preferred_element_type=f32) + b_ref[...])

    return pl.pallas_call(
        body,
        grid=(-(-n // BN),),
        in_specs=[
            pl.BlockSpec((NC, BN, D), lambda i: (0, i, 0)),
            pl.BlockSpec((NC, cnt2.shape[1]), lambda i: (0, 0)),
            pl.BlockSpec((D, dout), lambda i: (0, 0)),
            pl.BlockSpec((1, dout), lambda i: (0, 0)),
        ],
        out_specs=[
            pl.BlockSpec((BN, D), lambda i: (i, 0)),
            pl.BlockSpec((BN, dout), lambda i: (i, 0)),
        ],
        out_shape=[
            jax.ShapeDtypeStruct((n, D), f32),
            jax.ShapeDtypeStruct((n, dout), f32),
        ],
    )(s2, cnt2, w_fc, b_fc.reshape(1, dout))


def kernel(x_ent, x_cls, e1_src, e1_dst, e2_src, e2_dst, e3_src, e3_dst,
           W1_e1, b1_e1, W1_e2, b1_e2, W1_e3, b1_e3,
           W2_e1, b2_e1, W2_e2, b2_e2, W2_e3, b2_e3,
           W_fc, b_fc):
    n_ent = x_ent.shape[0]
    n_cls = x_cls.shape[0]
    rng_e, nr_e, ch_e = 12544, 4, 512
    rng_c, nr_c, ch_c = 10240, 1, 128

    t1 = _mm(x_ent, W1_e1, b1_e1)
    t3 = _mm(x_cls, W1_e3, b1_e3)

    s1b, d1b = _pad_edges(e1_src, e1_dst, n_ent)
    s3b, d3b = _pad_edges(e3_src, e3_dst, n_ent)
    s2b, d2b = _pad_edges(e2_src, e2_dst, n_cls)
    cnt1, cnt3, cnt2 = _counts([d1b, d3b, d2b], [ch_e, ch_e, ch_c])

    ob1, sz1 = _bucket(s1b, d1b, rng_e, nr_e)
    ob3, sz3 = _bucket(s3b, d3b, rng_e, nr_e)
    s1 = _seg_sum_b(ob1, sz1, t1, rng_e, nr_e)
    s3 = _seg_sum_b(ob3, sz3, t3, rng_e, nr_e)

    t2 = _combine1(s1, cnt1.reshape(NC, ch_e * EB),
                   s3, cnt3.reshape(NC, ch_e * EB), W2_e2, b2_e2,
                   rng_e * nr_e)

    s2 = _seg_sum(s2b, d2b, t2, rng_c, nr_c)

    output, pred = _combine2(s2, cnt2.reshape(NC, ch_c * EB), W_fc, b_fc, n_cls)
    return (output, pred)
